# Initial kernel scaffold; baseline (speedup 1.0000x reference)
import jax
import jax.numpy as jnp
from jax.experimental import pallas as pl


def kernel(positions, pe):
    raise NotImplementedError("write your pallas kernel here")


# speedup vs baseline: 1.4838x; 1.4838x over previous
import functools

import jax
import jax.numpy as jnp
from jax import lax
from jax.experimental import pallas as pl
from jax.experimental.pallas import tpu as pltpu
from jax.experimental.pallas import tpu_sc as plsc


def _sc_gather(n, D, chunk):
    info = plsc.get_sparse_core_info()
    nw = info.num_cores * info.num_subcores
    b_per_w = n // nw
    n_chunks = b_per_w // chunk
    mesh = plsc.VectorSubcoreMesh(core_axis_name="c", subcore_axis_name="s")

    @functools.partial(
        pl.kernel,
        out_type=jax.ShapeDtypeStruct((n, D), jnp.float32),
        mesh=mesh,
        scratch_types=[
            pltpu.VMEM((b_per_w,), jnp.int32),
            pltpu.VMEM((chunk, D), jnp.float32),
            pltpu.SemaphoreType.DMA,
        ],
    )
    def gather_kernel(pos_hbm, pe_hbm, out_hbm, idx_v, rows_v, sem):
        wid = lax.axis_index("s") * info.num_cores + lax.axis_index("c")
        base = wid * b_per_w
        pltpu.sync_copy(pos_hbm.at[pl.ds(base, b_per_w)], idx_v)

        @pl.loop(0, n_chunks)
        def _chunk(c):
            off = c * chunk
            pltpu.async_copy(
                pe_hbm.at[idx_v.at[pl.ds(off, chunk)]], rows_v, sem
            ).wait()
            pltpu.sync_copy(rows_v, out_hbm.at[pl.ds(base + off, chunk)])

    return gather_kernel


def kernel(positions, pe):
    B, S = positions.shape
    V, D = pe.shape
    n = B * S
    out = _sc_gather(n, D, chunk=32)(positions.reshape(n), pe)
    return out.reshape(B, S, D)


# speedup vs baseline: 1.6180x; 1.0904x over previous
import functools

import jax
import jax.numpy as jnp
from jax import lax
from jax.experimental import pallas as pl
from jax.experimental.pallas import tpu as pltpu
from jax.experimental.pallas import tpu_sc as plsc


def _sc_gather(n, D, chunk):
    info = plsc.get_sparse_core_info()
    nw = info.num_cores * info.num_subcores
    b_per_w = n // nw
    n_chunks = b_per_w // chunk
    assert n_chunks % 2 == 0
    mesh = plsc.VectorSubcoreMesh(core_axis_name="c", subcore_axis_name="s")

    @functools.partial(
        pl.kernel,
        out_type=jax.ShapeDtypeStruct((n, D), jnp.float32),
        mesh=mesh,
        scratch_types=[
            pltpu.VMEM((b_per_w,), jnp.int32),
            pltpu.VMEM((chunk, D), jnp.float32),
            pltpu.VMEM((chunk, D), jnp.float32),
            pltpu.SemaphoreType.DMA,
            pltpu.SemaphoreType.DMA,
        ],
    )
    def gather_kernel(pos_hbm, pe_hbm, out_hbm, idx_v, rows0, rows1, sem0, sem1):
        wid = lax.axis_index("s") * info.num_cores + lax.axis_index("c")
        base = wid * b_per_w
        pltpu.sync_copy(pos_hbm.at[pl.ds(base, b_per_w)], idx_v)

        bufs = (rows0, rows1)
        sems = (sem0, sem1)

        def start_gather(c, b):
            pltpu.make_async_copy(
                pe_hbm.at[idx_v.at[pl.ds(c * chunk, chunk)]], bufs[b], sems[b]
            ).start()

        def wait_gather(b):
            pltpu.make_async_copy(
                pe_hbm.at[idx_v.at[pl.ds(0, chunk)]], bufs[b], sems[b]
            ).wait()

        start_gather(0, 0)

        @pl.loop(0, n_chunks, step=2)
        def _pair(c0):
            start_gather(c0 + 1, 1)
            wait_gather(0)
            pltpu.sync_copy(bufs[0], out_hbm.at[pl.ds(base + c0 * chunk, chunk)])

            @pl.when(c0 + 2 < n_chunks)
            def _():
                start_gather(c0 + 2, 0)

            wait_gather(1)
            pltpu.sync_copy(
                bufs[1], out_hbm.at[pl.ds(base + (c0 + 1) * chunk, chunk)]
            )

    return gather_kernel


def kernel(positions, pe):
    B, S = positions.shape
    V, D = pe.shape
    n = B * S
    out = _sc_gather(n, D, chunk=16)(positions.reshape(n), pe)
    return out.reshape(B, S, D)
